# baseline (speedup 1.0000x reference)
import functools

import numpy as np
import jax
import jax.numpy as jnp
from jax import lax
from jax.experimental import pallas as pl
from jax.experimental.pallas import tpu as pltpu
from jax.experimental.pallas import tpu_sc as plsc

_C = 128
_NID = 6
_NC = 2
_NS = 16
_LANES = 16


def _build_consts():
    np.random.seed(0)
    m = np.zeros((_C, _NID), dtype=np.float32)
    m[:, 1:_NID] = np.random.rand(_C, _NID - 1)
    colT = m.T

    X = np.zeros(_NID, dtype=np.uint64)
    for i in range(_NID):
        for x in range(64):
            if (x >> i) & 1:
                X[i] |= np.uint64(1) << np.uint64(x)

    M = np.zeros(_NID, dtype=np.uint64)
    for j in range(_NID):
        for c in range(_C):
            pat = 0
            for i in range(_NID):
                if colT[i, c] >= colT[j, c]:
                    pat |= 1 << i
            M[j] |= np.uint64(1) << np.uint64(pat)

    def split(a):
        lo = (a & np.uint64(0xFFFFFFFF)).astype(np.uint32).view(np.int32)
        hi = (a >> np.uint64(32)).astype(np.uint32).view(np.int32)
        return [int(v) for v in lo], [int(v) for v in hi]

    xlo, xhi = split(X)
    mlo, mhi = split(M)
    return xlo, xhi, mlo, mhi


_XLO, _XHI, _MLO, _MHI = _build_consts()


def _pad16(vals):
    return np.asarray(vals + [0] * (_LANES - len(vals)), dtype=np.int32)


@functools.lru_cache(maxsize=None)
def _build_sc_call(nt, nb):
    nw = _NC * _NS
    bpw = nb // nw
    ngrp = bpw // _LANES
    mesh = plsc.VectorSubcoreMesh(core_axis_name="c", subcore_axis_name="s")

    @functools.partial(
        pl.kernel,
        mesh=mesh,
        out_type=jax.ShapeDtypeStruct((nb,), jnp.float32),
        scratch_types=[
            pltpu.VMEM((nt * bpw,), jnp.int32),
            pltpu.VMEM((nt * 2 * bpw,), jnp.float32),
            pltpu.VMEM((bpw,), jnp.int32),
            pltpu.VMEM((bpw,), jnp.float32),
            pltpu.VMEM((_LANES,), jnp.int32),
            pltpu.VMEM((_LANES,), jnp.int32),
            pltpu.VMEM((_LANES,), jnp.int32),
            pltpu.VMEM((_LANES,), jnp.int32),
            pltpu.SemaphoreType.DMA,
        ],
        compiler_params=pltpu.CompilerParams(needs_layout_passes=False),
    )
    def sc_trust(ids_hbm, perf_hbm, pred_hbm, xlo_hbm, xhi_hbm, mlo_hbm,
                 mhi_hbm, out_hbm, ids_v, perf_v, pred_v, out_v,
                 xlo_v, xhi_v, mlo_v, mhi_v, sem):
        cid = lax.axis_index("c")
        sid = lax.axis_index("s")
        wid = sid * _NC + cid
        base = wid * bpw

        def _slab_copy(t):
            a = pltpu.make_async_copy(
                ids_hbm.at[t, pl.ds(base, bpw)],
                ids_v.at[pl.ds(t * bpw, bpw)], sem)
            b = pltpu.make_async_copy(
                perf_hbm.at[t, pl.ds(2 * base, 2 * bpw)],
                perf_v.at[pl.ds(t * 2 * bpw, 2 * bpw)], sem)
            return a, b

        def stage(t, _):
            a, b = _slab_copy(t)
            a.start()
            b.start()
            return 0

        lax.fori_loop(0, nt, stage, 0)
        pltpu.sync_copy(pred_hbm.at[pl.ds(base, bpw)], pred_v)
        pltpu.sync_copy(xlo_hbm, xlo_v)
        pltpu.sync_copy(xhi_hbm, xhi_v)
        pltpu.sync_copy(mlo_hbm, mlo_v)
        pltpu.sync_copy(mhi_hbm, mhi_v)

        def drain(t, _):
            a, b = _slab_copy(t)
            a.wait()
            b.wait()
            return 0

        lax.fori_loop(0, nt, drain, 0)

        lanes2 = lax.iota(jnp.int32, _LANES) * 2
        zero_i = jnp.zeros((_LANES,), jnp.int32)
        neg1_i = jnp.full((_LANES,), -1, jnp.int32)
        glo_init = jnp.full((_LANES,), _XLO[0], jnp.int32)
        ghi_init = jnp.full((_LANES,), _XHI[0], jnp.int32)

        def outer(g, _):
            rb = g * _LANES

            def inner(t, carry):
                glo, ghi = carry
                id16 = ids_v[pl.ds(t * bpw + rb, _LANES)]
                pidx = lanes2 + (t * (2 * bpw) + 2 * rb)
                p0 = plsc.load_gather(perf_v, [pidx])
                p1 = plsc.load_gather(perf_v, [pidx + 1])
                b0 = p0 > 0.5
                b1 = p1 > 0.5
                s = jnp.logical_and(b1, jnp.logical_not(b0))
                f = jnp.logical_and(b0, jnp.logical_not(b1))
                xlo = plsc.load_gather(xlo_v, [id16])
                xhi = plsc.load_gather(xhi_v, [id16])
                glo = (glo | jnp.where(s, xlo, zero_i)) & jnp.where(f, xlo, neg1_i)
                ghi = (ghi | jnp.where(s, xhi, zero_i)) & jnp.where(f, xhi, neg1_i)
                return glo, ghi

            glo, ghi = lax.fori_loop(0, nt, inner, (glo_init, ghi_init))

            pred16 = pred_v[pl.ds(rb, _LANES)]
            mlo = plsc.load_gather(mlo_v, [pred16])
            mhi = plsc.load_gather(mhi_v, [pred16])
            ok = jnp.logical_and((glo & mlo) == mlo, (ghi & mhi) == mhi)
            out_v[pl.ds(rb, _LANES)] = jnp.where(ok, 1.0, 0.0).astype(jnp.float32)
            return 0

        lax.fori_loop(0, ngrp, outer, 0)
        pltpu.sync_copy(out_v, out_hbm.at[pl.ds(base, bpw)])

    return sc_trust


def kernel(inptasksobs, inptasksperf, inptaskspred, num_obs_tasks, tasksobsids, taskspredids):
    nt = tasksobsids.shape[0]
    nb = tasksobsids.shape[1]

    ids = tasksobsids.reshape(nt, nb)
    perf = inptasksperf.reshape(nt, 2 * nb)
    pred = taskspredids.reshape(nb)

    xlo = jnp.asarray(_pad16(_XLO))
    xhi = jnp.asarray(_pad16(_XHI))
    mlo = jnp.asarray(_pad16(_MLO))
    mhi = jnp.asarray(_pad16(_MHI))

    trust = _build_sc_call(nt, nb)(ids, perf, pred, xlo, xhi, mlo, mhi)
    return trust.reshape(nb, 1)


# speedup vs baseline: 1.1093x; 1.1093x over previous
import functools

import numpy as np
import jax
import jax.numpy as jnp
from jax import lax
from jax.experimental import pallas as pl
from jax.experimental.pallas import tpu as pltpu
from jax.experimental.pallas import tpu_sc as plsc

_C = 128
_NID = 6
_NC = 2
_NS = 16
_LANES = 16
_GUNROLL = 8


def _build_consts():
    np.random.seed(0)
    m = np.zeros((_C, _NID), dtype=np.float32)
    m[:, 1:_NID] = np.random.rand(_C, _NID - 1)
    colT = m.T

    X = np.zeros(_NID, dtype=np.uint32)
    for i in range(_NID):
        for y in range(32):
            if ((2 * y) >> i) & 1:
                X[i] |= np.uint32(1) << np.uint32(y)

    M = np.zeros(_NID, dtype=np.uint32)
    for j in range(1, _NID):
        for c in range(_C):
            pat = 0
            for i in range(_NID):
                if colT[i, c] >= colT[j, c]:
                    pat |= 1 << i
            M[j] |= np.uint32(1) << np.uint32(pat // 2)

    return ([int(v) for v in X.view(np.int32)],
            [int(v) for v in M.view(np.int32)])


_X32, _M32 = _build_consts()


def _pad16(vals):
    return np.asarray(vals + [0] * (_LANES - len(vals)), dtype=np.int32)


@functools.lru_cache(maxsize=None)
def _build_sc_call(nt, nb):
    nw = _NC * _NS
    bpw = nb // nw
    ngrp = bpw // _LANES
    nblk = ngrp // _GUNROLL
    mesh = plsc.VectorSubcoreMesh(core_axis_name="c", subcore_axis_name="s")

    @functools.partial(
        pl.kernel,
        mesh=mesh,
        out_type=jax.ShapeDtypeStruct((nb,), jnp.float32),
        scratch_types=[
            pltpu.VMEM((nt * bpw,), jnp.int32),
            pltpu.VMEM((nt * 2 * bpw,), jnp.float32),
            pltpu.VMEM((bpw,), jnp.int32),
            pltpu.VMEM((bpw,), jnp.float32),
            pltpu.VMEM((_LANES,), jnp.int32),
            pltpu.VMEM((_LANES,), jnp.int32),
            pltpu.SemaphoreType.DMA,
        ],
        compiler_params=pltpu.CompilerParams(needs_layout_passes=False),
    )
    def sc_trust(ids_hbm, perf_hbm, pred_hbm, x32_hbm, m32_hbm,
                 out_hbm, ids_v, perf_v, pred_v, out_v, x32_v, m32_v, sem):
        cid = lax.axis_index("c")
        sid = lax.axis_index("s")
        wid = sid * _NC + cid
        base = wid * bpw

        def _slab_copy(t):
            a = pltpu.make_async_copy(
                ids_hbm.at[t, pl.ds(base, bpw)],
                ids_v.at[pl.ds(t * bpw, bpw)], sem)
            b = pltpu.make_async_copy(
                perf_hbm.at[t, pl.ds(2 * base, 2 * bpw)],
                perf_v.at[pl.ds(t * 2 * bpw, 2 * bpw)], sem)
            return a, b

        def stage(t, _):
            a, b = _slab_copy(t)
            a.start()
            b.start()
            return 0

        lax.fori_loop(0, nt, stage, 0)
        pltpu.sync_copy(pred_hbm.at[pl.ds(base, bpw)], pred_v)
        pltpu.sync_copy(x32_hbm, x32_v)
        pltpu.sync_copy(m32_hbm, m32_v)

        def drain(t, _):
            a, b = _slab_copy(t)
            a.wait()
            b.wait()
            return 0

        lax.fori_loop(0, nt, drain, 0)

        lanes2 = lax.iota(jnp.int32, _LANES) * 2
        zero_i = jnp.zeros((_LANES,), jnp.int32)
        neg1_i = jnp.full((_LANES,), -1, jnp.int32)
        half = jnp.float32(0.5)

        def gblock(gb, _):
            rb0 = gb * (_GUNROLL * _LANES)
            pbase = [lanes2 + (rb0 * 2 + u * (2 * _LANES))
                     for u in range(_GUNROLL)]

            def inner(t, carry):
                tb_ids = t * bpw + rb0
                tb_perf = t * (2 * bpw)
                out = []
                for u in range(_GUNROLL):
                    g = carry[u]
                    id16 = ids_v[pl.ds(tb_ids + u * _LANES, _LANES)]
                    pidx = pbase[u] + tb_perf
                    p0 = plsc.load_gather(perf_v, [pidx])
                    p1 = plsc.load_gather(perf_v, [pidx + 1])
                    x = plsc.load_gather(x32_v, [id16])
                    b0 = p0 > half
                    b1 = p1 > half
                    s = jnp.logical_and(b1, jnp.logical_not(b0))
                    f = jnp.logical_and(b0, jnp.logical_not(b1))
                    g = (g | jnp.where(s, x, zero_i)) & jnp.where(f, x, neg1_i)
                    out.append(g)
                return tuple(out)

            gs = lax.fori_loop(0, nt, inner,
                               tuple(zero_i for _ in range(_GUNROLL)))

            for u in range(_GUNROLL):
                pred16 = pred_v[pl.ds(rb0 + u * _LANES, _LANES)]
                m = plsc.load_gather(m32_v, [pred16])
                ok = (gs[u] & m) == m
                out_v[pl.ds(rb0 + u * _LANES, _LANES)] = (
                    jnp.where(ok, 1.0, 0.0).astype(jnp.float32))
            return 0

        lax.fori_loop(0, nblk, gblock, 0)
        pltpu.sync_copy(out_v, out_hbm.at[pl.ds(base, bpw)])

    return sc_trust


def kernel(inptasksobs, inptasksperf, inptaskspred, num_obs_tasks, tasksobsids, taskspredids):
    nt = tasksobsids.shape[0]
    nb = tasksobsids.shape[1]

    ids = tasksobsids.reshape(nt, nb)
    perf = inptasksperf.reshape(nt, 2 * nb)
    pred = taskspredids.reshape(nb)

    x32 = jnp.asarray(_pad16(_X32))
    m32 = jnp.asarray(_pad16(_M32))

    trust = _build_sc_call(nt, nb)(ids, perf, pred, x32, m32)
    return trust.reshape(nb, 1)


# speedup vs baseline: 1.1600x; 1.0457x over previous
import functools

import numpy as np
import jax
import jax.numpy as jnp
from jax import lax
from jax.experimental import pallas as pl
from jax.experimental.pallas import tpu as pltpu
from jax.experimental.pallas import tpu_sc as plsc

_C = 128
_NID = 6
_NC = 2
_NS = 16
_LANES = 16
_GUNROLL = 8


def _build_consts():
    np.random.seed(0)
    m = np.zeros((_C, _NID), dtype=np.float32)
    m[:, 1:_NID] = np.random.rand(_C, _NID - 1)
    colT = m.T

    X = np.zeros(_NID, dtype=np.uint32)
    for i in range(_NID):
        for y in range(32):
            if ((2 * y) >> i) & 1:
                X[i] |= np.uint32(1) << np.uint32(y)

    M = np.zeros(_NID, dtype=np.uint32)
    for j in range(1, _NID):
        for c in range(_C):
            pat = 0
            for i in range(_NID):
                if colT[i, c] >= colT[j, c]:
                    pat |= 1 << i
            M[j] |= np.uint32(1) << np.uint32(pat // 2)

    return ([int(v) for v in X.view(np.int32)],
            [int(v) for v in M.view(np.int32)])


_X32, _M32 = _build_consts()


def _pad16(vals):
    return np.asarray(vals + [0] * (_LANES - len(vals)), dtype=np.int32)


@functools.lru_cache(maxsize=None)
def _build_sc_call(nt, nb):
    nw = _NC * _NS
    bpw = nb // nw
    ngrp = bpw // _LANES
    nblk = ngrp // _GUNROLL
    mesh = plsc.VectorSubcoreMesh(core_axis_name="c", subcore_axis_name="s")

    @functools.partial(
        pl.kernel,
        mesh=mesh,
        out_type=jax.ShapeDtypeStruct((nb,), jnp.float32),
        scratch_types=[
            pltpu.VMEM((nt * bpw,), jnp.int32),
            pltpu.VMEM((nt * bpw,), jnp.float32),
            pltpu.VMEM((nt * bpw,), jnp.float32),
            pltpu.VMEM((bpw,), jnp.int32),
            pltpu.VMEM((bpw,), jnp.float32),
            pltpu.VMEM((_LANES,), jnp.int32),
            pltpu.VMEM((_LANES,), jnp.int32),
            pltpu.SemaphoreType.DMA,
        ],
        compiler_params=pltpu.CompilerParams(needs_layout_passes=False),
    )
    def sc_trust(ids_hbm, p0_hbm, p1_hbm, pred_hbm, x32_hbm, m32_hbm,
                 out_hbm, ids_v, p0_v, p1_v, pred_v, out_v, x32_v, m32_v, sem):
        cid = lax.axis_index("c")
        sid = lax.axis_index("s")
        wid = sid * _NC + cid
        base = wid * bpw

        def _slab_copy(t):
            a = pltpu.make_async_copy(
                ids_hbm.at[t, pl.ds(base, bpw)],
                ids_v.at[pl.ds(t * bpw, bpw)], sem)
            b = pltpu.make_async_copy(
                p0_hbm.at[t, pl.ds(base, bpw)],
                p0_v.at[pl.ds(t * bpw, bpw)], sem)
            c = pltpu.make_async_copy(
                p1_hbm.at[t, pl.ds(base, bpw)],
                p1_v.at[pl.ds(t * bpw, bpw)], sem)
            return a, b, c

        def stage(t, _):
            a, b, c = _slab_copy(t)
            a.start()
            b.start()
            c.start()
            return 0

        lax.fori_loop(0, nt, stage, 0)
        pltpu.sync_copy(pred_hbm.at[pl.ds(base, bpw)], pred_v)
        pltpu.sync_copy(x32_hbm, x32_v)
        pltpu.sync_copy(m32_hbm, m32_v)

        def drain(t, _):
            a, b, c = _slab_copy(t)
            a.wait()
            b.wait()
            c.wait()
            return 0

        lax.fori_loop(0, nt, drain, 0)

        zero_i = jnp.zeros((_LANES,), jnp.int32)
        neg1_i = jnp.full((_LANES,), -1, jnp.int32)
        half = jnp.float32(0.5)

        def gblock(gb, _):
            rb0 = gb * (_GUNROLL * _LANES)

            def inner(t, carry):
                tb = t * bpw + rb0
                out = []
                for u in range(_GUNROLL):
                    g = carry[u]
                    id16 = ids_v[pl.ds(tb + u * _LANES, _LANES)]
                    p0 = p0_v[pl.ds(tb + u * _LANES, _LANES)]
                    p1 = p1_v[pl.ds(tb + u * _LANES, _LANES)]
                    x = plsc.load_gather(x32_v, [id16])
                    b0 = p0 > half
                    b1 = p1 > half
                    s = jnp.logical_and(b1, jnp.logical_not(b0))
                    f = jnp.logical_and(b0, jnp.logical_not(b1))
                    g = (g | jnp.where(s, x, zero_i)) & jnp.where(f, x, neg1_i)
                    out.append(g)
                return tuple(out)

            gs = lax.fori_loop(0, nt, inner,
                               tuple(zero_i for _ in range(_GUNROLL)))

            for u in range(_GUNROLL):
                pred16 = pred_v[pl.ds(rb0 + u * _LANES, _LANES)]
                m = plsc.load_gather(m32_v, [pred16])
                ok = (gs[u] & m) == m
                out_v[pl.ds(rb0 + u * _LANES, _LANES)] = (
                    jnp.where(ok, 1.0, 0.0).astype(jnp.float32))
            return 0

        lax.fori_loop(0, nblk, gblock, 0)
        pltpu.sync_copy(out_v, out_hbm.at[pl.ds(base, bpw)])

    return sc_trust


def kernel(inptasksobs, inptasksperf, inptaskspred, num_obs_tasks, tasksobsids, taskspredids):
    nt = tasksobsids.shape[0]
    nb = tasksobsids.shape[1]

    x32 = jnp.asarray(_pad16(_X32))
    m32 = jnp.asarray(_pad16(_M32))

    ids = tasksobsids[:, :, 0]
    p0 = inptasksperf[:, :, 0]
    p1 = inptasksperf[:, :, 1]
    pred = taskspredids[:, 0]

    trust = _build_sc_call(nt, nb)(ids, p0, p1, pred, x32, m32)
    return trust.reshape(nb, 1)


# speedup vs baseline: 2.9211x; 2.5182x over previous
import numpy as np
import jax
import jax.numpy as jnp
from jax import lax
from jax.experimental import pallas as pl
from jax.experimental.pallas import tpu as pltpu

_C = 128
_NID = 6


def _build_consts():
    np.random.seed(0)
    m = np.zeros((_C, _NID), dtype=np.float32)
    m[:, 1:_NID] = np.random.rand(_C, _NID - 1)
    colT = m.T

    X = np.zeros(_NID, dtype=np.uint32)
    for i in range(_NID):
        for y in range(32):
            if ((2 * y) >> i) & 1:
                X[i] |= np.uint32(1) << np.uint32(y)

    M = np.zeros(_NID, dtype=np.uint32)
    for j in range(1, _NID):
        for c in range(_C):
            pat = 0
            for i in range(_NID):
                if colT[i, c] >= colT[j, c]:
                    pat |= 1 << i
            M[j] |= np.uint32(1) << np.uint32(pat // 2)

    return ([int(v) for v in X.view(np.int32)],
            [int(v) for v in M.view(np.int32)])


_X32, _M32 = _build_consts()


def _select6(idx, consts):
    out = jnp.full(idx.shape, consts[0], dtype=jnp.int32)
    for i in range(1, _NID):
        out = jnp.where(idx == i, jnp.int32(consts[i]), out)
    return out


def _trust_body(ids_ref, p0_ref, p1_ref, pred_ref, out_ref):
    nt = ids_ref.shape[0]
    shp = ids_ref.shape[1:]
    zero = jnp.zeros(shp, dtype=jnp.int32)
    neg1 = jnp.full(shp, -1, dtype=jnp.int32)

    def step(t, g):
        idt = ids_ref[t]
        p0 = p0_ref[t] > 0.5
        p1 = p1_ref[t] > 0.5
        s = jnp.logical_and(p1, jnp.logical_not(p0))
        f = jnp.logical_and(p0, jnp.logical_not(p1))
        x = _select6(idt, _X32)
        return (g | jnp.where(s, x, zero)) & jnp.where(f, x, neg1)

    g = lax.fori_loop(0, nt, step, zero, unroll=True)

    m = _select6(pred_ref[...], _M32)
    out_ref[...] = ((g & m) == m).astype(jnp.float32)


def kernel(inptasksobs, inptasksperf, inptaskspred, num_obs_tasks, tasksobsids, taskspredids):
    nt = tasksobsids.shape[0]
    nb = tasksobsids.shape[1]
    lanes = 128
    rows = nb // lanes

    ids = tasksobsids.reshape(nt, rows, lanes)
    p0 = inptasksperf[..., 0].reshape(nt, rows, lanes)
    p1 = inptasksperf[..., 1].reshape(nt, rows, lanes)
    pred = taskspredids.reshape(rows, lanes)

    rblk = 16
    grid = (rows // rblk,)
    trust = pl.pallas_call(
        _trust_body,
        grid=grid,
        in_specs=[
            pl.BlockSpec((nt, rblk, lanes), lambda r: (0, r, 0)),
            pl.BlockSpec((nt, rblk, lanes), lambda r: (0, r, 0)),
            pl.BlockSpec((nt, rblk, lanes), lambda r: (0, r, 0)),
            pl.BlockSpec((rblk, lanes), lambda r: (r, 0)),
        ],
        out_specs=pl.BlockSpec((rblk, lanes), lambda r: (r, 0)),
        out_shape=jax.ShapeDtypeStruct((rows, lanes), jnp.float32),
        compiler_params=pltpu.CompilerParams(
            allow_input_fusion=[True, True, True, True]),
    )(ids, p0, p1, pred)

    return trust.reshape(nb, 1)
